# baseline (speedup 1.0000x reference)
import functools

import jax
import jax.numpy as jnp
from jax import lax
from jax.experimental import pallas as pl
from jax.experimental.pallas import tpu as pltpu
from jax.experimental.pallas import tpu_sc as plsc

N_NODES = 10000
D = 256
DH = 128
EPS = 1e-5

NC = 2
NS = 16
CHUNK = 128
NCH = 80
SEC = 16
NSEC = NCH // SEC
EPT = CHUNK * NCH
E_PAD = EPT * NS
ACC_ROWS = 10112
ROWS_PT = ACC_ROWS // NS

ACH_CHUNK = 128
ACH = 80

_mesh = plsc.VectorSubcoreMesh(core_axis_name="c", subcore_axis_name="s",
                               num_cores=NC, num_subcores=NS)


def _degrees_body(eh_ref, zero_ref, out_ref, idx_v, vals_v, hist_sh, sem):
    c = lax.axis_index("c")
    s = lax.axis_index("s")
    ones16 = jnp.ones((16,), jnp.float32)
    for r in range(ACH_CHUNK):
        vals_v[r, :] = ones16
    pltpu.sync_copy(zero_ref.at[pl.ds(s * ROWS_PT, ROWS_PT)],
                    hist_sh.at[pl.ds(s * ROWS_PT, ROWS_PT)])
    plsc.subcore_barrier()
    pltpu.sync_copy(eh_ref.at[c, s], idx_v)

    def group(g, carry):
        descs = []
        for k in range(8):
            descs.append(
                pltpu.async_copy(vals_v, hist_sh.at[idx_v.at[g * 8 + k]],
                                 sem, add=True))
        for d in descs:
            d.wait()
        return carry

    lax.fori_loop(0, ACH // 8, group, 0)
    plsc.subcore_barrier()
    pltpu.sync_copy(hist_sh.at[pl.ds(s * ROWS_PT, ROWS_PT)],
                    out_ref.at[c, pl.ds(s * ROWS_PT, ROWS_PT)])


_degrees_kernel = pl.kernel(
    _degrees_body,
    out_type=jax.ShapeDtypeStruct((NC, ACC_ROWS, 16), jnp.float32),
    mesh=_mesh,
    scratch_types=[
        pltpu.VMEM((ACH, ACH_CHUNK), jnp.int32),
        pltpu.VMEM((ACH_CHUNK, 16), jnp.float32),
        pltpu.VMEM_SHARED((ACC_ROWS, 16), jnp.float32),
        pltpu.SemaphoreType.DMA,
    ],
    compiler_params=pltpu.CompilerParams(use_tc_tiling_on_sc=False),
)


def _ln_scale_body(h_ref, deg_ref, g_ref, b_ref, out_ref):
    h = h_ref[...]
    mean = jnp.mean(h, axis=-1, keepdims=True)
    var = jnp.mean((h - mean) ** 2, axis=-1, keepdims=True)
    hn = (h - mean) * lax.rsqrt(var + EPS) * g_ref[...] + b_ref[...]
    deg = deg_ref[0, :, 0:1]
    ns = jnp.where(deg > 0, lax.rsqrt(jnp.maximum(deg, 1.0)), 0.0)
    sc = hn * ns
    out_ref[0, :, :] = sc[:, :DH]
    out_ref[1, :, :] = sc[:, DH:]


def _edge_body(tab_ref, src_ref, dst_ref, out_ref,
               sidx_v, didx_v, rows_v, zb_v, acc_sh, gsem, ssem, zsem):
    c = lax.axis_index("c")
    s = lax.axis_index("s")

    z16 = jnp.zeros((16,), jnp.float32)

    def zrow(rr, carry):
        for l in range(DH // 16):
            zb_v[rr, pl.ds(l * 16, 16)] = z16
        return carry

    lax.fori_loop(0, 16, zrow, 0)

    zd = []
    for t in range(39):
        zd.append(pltpu.async_copy(
            zb_v, acc_sh.at[pl.ds(s * ROWS_PT + t * 16, 16)], zsem))
    zd.append(pltpu.async_copy(
        zb_v.at[pl.ds(0, 8)],
        acc_sh.at[pl.ds(s * ROWS_PT + 624, 8)], zsem))
    pltpu.sync_copy(src_ref.at[c, s], sidx_v)
    for d in zd:
        d.wait()
    plsc.subcore_barrier()

    def gather(j, slot):
        pltpu.async_copy(tab_ref.at[sidx_v.at[j]], rows_v.at[slot], gsem)

    def gwait(j, slot):
        pltpu.make_async_copy(tab_ref.at[sidx_v.at[j]],
                              rows_v.at[slot], gsem).wait()

    def swait(l, slot):
        pltpu.make_async_copy(rows_v.at[slot],
                              acc_sh.at[didx_v.at[l]], ssem).wait()

    gather(0, 0)

    def section(sec, carry):
        @pl.when(sec > 0)
        def _():
            swait(SEC - 1, 1)
        pltpu.sync_copy(dst_ref.at[s, pl.ds(sec * SEC, SEC)], didx_v)

        def do_chunk(l, b, drain):
            j = sec * SEC + l
            gwait(j, b)
            pltpu.async_copy(rows_v.at[b], acc_sh.at[didx_v.at[l]],
                             ssem, add=True)
            if drain:
                swait(l - 1, 1 - b)

            @pl.when(j + 1 < NCH)
            def _():
                gather(j + 1, 1 - b)

        do_chunk(0, 0, False)
        do_chunk(1, 1, True)

        def step(ii, carry2):
            for b in range(2):
                do_chunk(ii * 2 + b, b, True)
            return carry2

        lax.fori_loop(1, SEC // 2, step, 0)
        return carry

    lax.fori_loop(0, NSEC, section, 0)
    swait(SEC - 1, 1)
    plsc.subcore_barrier()
    pltpu.sync_copy(acc_sh.at[pl.ds(s * ROWS_PT, ROWS_PT)],
                    out_ref.at[c, pl.ds(s * ROWS_PT, ROWS_PT)])


_edge_kernel = pl.kernel(
    _edge_body,
    out_type=jax.ShapeDtypeStruct((NC, ACC_ROWS, DH), jnp.float32),
    mesh=_mesh,
    scratch_types=[
        pltpu.VMEM((NCH, CHUNK), jnp.int32),
        pltpu.VMEM((SEC, CHUNK), jnp.int32),
        pltpu.VMEM((2, CHUNK, DH), jnp.float32),
        pltpu.VMEM((16, DH), jnp.float32),
        pltpu.VMEM_SHARED((ACC_ROWS, DH), jnp.float32),
        pltpu.SemaphoreType.DMA,
        pltpu.SemaphoreType.DMA,
        pltpu.SemaphoreType.DMA,
    ],
    compiler_params=pltpu.CompilerParams(use_tc_tiling_on_sc=True),
)


def _ffn_body(agg_ref, deg_ref, wt_ref, b_ref, out_ref):
    deg = deg_ref[0, :, 0:1]
    nd = jnp.where(deg > 0, lax.rsqrt(jnp.maximum(deg, 1.0)), 0.0)
    hc = jnp.concatenate([agg_ref[0, :, :] * nd, agg_ref[1, :, :] * nd],
                         axis=-1)
    out_ref[...] = jnp.dot(hc, wt_ref[...],
                           preferred_element_type=jnp.float32) + b_ref[...]


def kernel(h, edge_index, gamma, beta, W, b):
    N = h.shape[0]
    E = edge_index.shape[1]
    pad = E_PAD - E

    src = edge_index[0]
    dst = edge_index[1]

    eh = jnp.concatenate(
        [edge_index, jnp.full((2, pad), N, jnp.int32)], axis=1
    ).reshape(NC, NS, ACH, ACH_CHUNK)
    zero16 = jnp.zeros((ACC_ROWS, 16), jnp.float32)
    degs = _degrees_kernel(eh, zero16)

    nb = 10
    rows = N // nb
    hn2 = pl.pallas_call(
        _ln_scale_body,
        grid=(nb,),
        in_specs=[
            pl.BlockSpec((rows, D), lambda r: (r, 0)),
            pl.BlockSpec((1, rows, 16), lambda r: (0, r, 0)),
            pl.BlockSpec((1, D), lambda r: (0, 0)),
            pl.BlockSpec((1, D), lambda r: (0, 0)),
        ],
        out_specs=pl.BlockSpec((NC, rows, DH), lambda r: (0, r, 0)),
        out_shape=jax.ShapeDtypeStruct((NC, N, DH), jnp.float32),
    )(h, degs, gamma.reshape(1, D), beta.reshape(1, D))

    src_p = jnp.concatenate([src, jnp.zeros((pad,), jnp.int32)])
    dst_p = jnp.concatenate([dst, jnp.full((pad,), N, jnp.int32)])
    offs = jnp.arange(NC, dtype=jnp.int32).reshape(NC, 1) * N
    srcC = (src_p[None, :] + offs).reshape(NC, NS, NCH, CHUNK)
    dstC = dst_p.reshape(NS, NCH, CHUNK)
    agg = _edge_kernel(hn2.reshape(NC * N, DH), srcC, dstC)

    out = pl.pallas_call(
        _ffn_body,
        grid=(nb,),
        in_specs=[
            pl.BlockSpec((NC, rows, DH), lambda r: (0, r, 0)),
            pl.BlockSpec((1, rows, 16), lambda r: (1, r, 0)),
            pl.BlockSpec((D, D), lambda r: (0, 0)),
            pl.BlockSpec((1, D), lambda r: (0, 0)),
        ],
        out_specs=pl.BlockSpec((rows, D), lambda r: (r, 0)),
        out_shape=jax.ShapeDtypeStruct((N, D), jnp.float32),
    )(agg, degs, W.T, b.reshape(1, D))
    return out


# speedup vs baseline: 1.0397x; 1.0397x over previous
import functools

import jax
import jax.numpy as jnp
from jax import lax
from jax.experimental import pallas as pl
from jax.experimental.pallas import tpu as pltpu
from jax.experimental.pallas import tpu_sc as plsc

N_NODES = 10000
D = 256
DH = 128
EPS = 1e-5

NC = 2
NS = 16
CHUNK = 64
NCH = 160
SEC = 32
NSEC = NCH // SEC
NSLOT = 4
EPT = CHUNK * NCH
E_PAD = EPT * NS
ACC_ROWS = 10112
ROWS_PT = ACC_ROWS // NS

ACH_CHUNK = 128
ACH = 80

_mesh = plsc.VectorSubcoreMesh(core_axis_name="c", subcore_axis_name="s",
                               num_cores=NC, num_subcores=NS)


def _degrees_body(eh_ref, zero_ref, out_ref, idx_v, vals_v, hist_sh, sem):
    c = lax.axis_index("c")
    s = lax.axis_index("s")
    ones16 = jnp.ones((16,), jnp.float32)
    for r in range(ACH_CHUNK):
        vals_v[r, :] = ones16
    pltpu.sync_copy(zero_ref.at[pl.ds(s * ROWS_PT, ROWS_PT)],
                    hist_sh.at[pl.ds(s * ROWS_PT, ROWS_PT)])
    plsc.subcore_barrier()
    pltpu.sync_copy(eh_ref.at[c, s], idx_v)

    def group(g, carry):
        descs = []
        for k in range(8):
            descs.append(
                pltpu.async_copy(vals_v, hist_sh.at[idx_v.at[g * 8 + k]],
                                 sem, add=True))
        for d in descs:
            d.wait()
        return carry

    lax.fori_loop(0, ACH // 8, group, 0)
    plsc.subcore_barrier()
    pltpu.sync_copy(hist_sh.at[pl.ds(s * ROWS_PT, ROWS_PT)],
                    out_ref.at[c, pl.ds(s * ROWS_PT, ROWS_PT)])


_degrees_kernel = pl.kernel(
    _degrees_body,
    out_type=jax.ShapeDtypeStruct((NC, ACC_ROWS, 16), jnp.float32),
    mesh=_mesh,
    scratch_types=[
        pltpu.VMEM((ACH, ACH_CHUNK), jnp.int32),
        pltpu.VMEM((ACH_CHUNK, 16), jnp.float32),
        pltpu.VMEM_SHARED((ACC_ROWS, 16), jnp.float32),
        pltpu.SemaphoreType.DMA,
    ],
    compiler_params=pltpu.CompilerParams(use_tc_tiling_on_sc=False),
)


def _ln_scale_body(h_ref, deg_ref, g_ref, b_ref, out_ref):
    h = h_ref[...]
    mean = jnp.mean(h, axis=-1, keepdims=True)
    var = jnp.mean((h - mean) ** 2, axis=-1, keepdims=True)
    hn = (h - mean) * lax.rsqrt(var + EPS) * g_ref[...] + b_ref[...]
    deg = deg_ref[0, :, 0:1]
    ns = jnp.where(deg > 0, lax.rsqrt(jnp.maximum(deg, 1.0)), 0.0)
    sc = hn * ns
    out_ref[0, :, :] = sc[:, :DH]
    out_ref[1, :, :] = sc[:, DH:]


def _edge_body(tab_ref, src_ref, dst_ref, out_ref,
               sidx_v, didx_v, rows_v, zb_v, acc_sh, gsem, ssem, zsem):
    c = lax.axis_index("c")
    s = lax.axis_index("s")

    z16 = jnp.zeros((16,), jnp.float32)

    def zrow(rr, carry):
        for l in range(DH // 16):
            zb_v[rr, pl.ds(l * 16, 16)] = z16
        return carry

    lax.fori_loop(0, 16, zrow, 0)

    zd = []
    for t in range(39):
        zd.append(pltpu.async_copy(
            zb_v, acc_sh.at[pl.ds(s * ROWS_PT + t * 16, 16)], zsem))
    zd.append(pltpu.async_copy(
        zb_v.at[pl.ds(0, 8)],
        acc_sh.at[pl.ds(s * ROWS_PT + 624, 8)], zsem))
    pltpu.sync_copy(src_ref.at[c, s], sidx_v)
    for d in zd:
        d.wait()
    plsc.subcore_barrier()

    def gather(j, slot):
        pltpu.async_copy(tab_ref.at[sidx_v.at[j]], rows_v.at[slot], gsem)

    def gwait(j, slot):
        pltpu.make_async_copy(tab_ref.at[sidx_v.at[j]],
                              rows_v.at[slot], gsem).wait()

    def swait(l, slot):
        pltpu.make_async_copy(rows_v.at[slot],
                              acc_sh.at[didx_v.at[l]], ssem).wait()

    for k in range(NSLOT):
        gather(k, k)

    def section(sec, carry):
        pltpu.sync_copy(dst_ref.at[s, pl.ds(sec * SEC, SEC)], didx_v)

        def step(ii, carry2):
            for b in range(NSLOT):
                l = ii * NSLOT + b
                j = sec * SEC + l
                gwait(j, b)
                pltpu.async_copy(rows_v.at[b], acc_sh.at[didx_v.at[l]],
                                 ssem, add=True)

                @pl.when(j + NSLOT < NCH)
                def _():
                    swait(l, b)
                    gather(j + NSLOT, b)
            return carry2

        lax.fori_loop(0, SEC // NSLOT, step, 0)
        return carry

    lax.fori_loop(0, NSEC, section, 0)
    for k in range(NSLOT):
        swait(SEC - NSLOT + k, k)
    plsc.subcore_barrier()
    pltpu.sync_copy(acc_sh.at[pl.ds(s * ROWS_PT, ROWS_PT)],
                    out_ref.at[c, pl.ds(s * ROWS_PT, ROWS_PT)])


_edge_kernel = pl.kernel(
    _edge_body,
    out_type=jax.ShapeDtypeStruct((NC, ACC_ROWS, DH), jnp.float32),
    mesh=_mesh,
    scratch_types=[
        pltpu.VMEM((NCH, CHUNK), jnp.int32),
        pltpu.VMEM((SEC, CHUNK), jnp.int32),
        pltpu.VMEM((NSLOT, CHUNK, DH), jnp.float32),
        pltpu.VMEM((16, DH), jnp.float32),
        pltpu.VMEM_SHARED((ACC_ROWS, DH), jnp.float32),
        pltpu.SemaphoreType.DMA,
        pltpu.SemaphoreType.DMA,
        pltpu.SemaphoreType.DMA,
    ],
    compiler_params=pltpu.CompilerParams(use_tc_tiling_on_sc=False),
)


def _ffn_body(agg_ref, deg_ref, wt_ref, b_ref, out_ref):
    deg = deg_ref[0, :, 0:1]
    nd = jnp.where(deg > 0, lax.rsqrt(jnp.maximum(deg, 1.0)), 0.0)
    hc = jnp.concatenate([agg_ref[0, :, :] * nd, agg_ref[1, :, :] * nd],
                         axis=-1)
    out_ref[...] = jnp.dot(hc, wt_ref[...],
                           preferred_element_type=jnp.float32) + b_ref[...]


def kernel(h, edge_index, gamma, beta, W, b):
    N = h.shape[0]
    E = edge_index.shape[1]
    pad = E_PAD - E

    src = edge_index[0]
    dst = edge_index[1]

    eh = jnp.concatenate(
        [edge_index, jnp.full((2, pad), N, jnp.int32)], axis=1
    ).reshape(NC, NS, ACH, ACH_CHUNK)
    zero16 = jnp.zeros((ACC_ROWS, 16), jnp.float32)
    degs = _degrees_kernel(eh, zero16)

    nb = 10
    rows = N // nb
    hn2 = pl.pallas_call(
        _ln_scale_body,
        grid=(nb,),
        in_specs=[
            pl.BlockSpec((rows, D), lambda r: (r, 0)),
            pl.BlockSpec((1, rows, 16), lambda r: (0, r, 0)),
            pl.BlockSpec((1, D), lambda r: (0, 0)),
            pl.BlockSpec((1, D), lambda r: (0, 0)),
        ],
        out_specs=pl.BlockSpec((NC, rows, DH), lambda r: (0, r, 0)),
        out_shape=jax.ShapeDtypeStruct((NC, N, DH), jnp.float32),
    )(h, degs, gamma.reshape(1, D), beta.reshape(1, D))

    src_p = jnp.concatenate([src, jnp.zeros((pad,), jnp.int32)])
    dst_p = jnp.concatenate([dst, jnp.full((pad,), N, jnp.int32)])
    offs = jnp.arange(NC, dtype=jnp.int32).reshape(NC, 1) * N
    srcC = (src_p[None, :] + offs).reshape(NC, NS, NCH, CHUNK)
    dstC = dst_p.reshape(NS, NCH, CHUNK)
    agg = _edge_kernel(hn2.reshape(NC * N, DH), srcC, dstC)

    out = pl.pallas_call(
        _ffn_body,
        grid=(nb,),
        in_specs=[
            pl.BlockSpec((NC, rows, DH), lambda r: (0, r, 0)),
            pl.BlockSpec((1, rows, 16), lambda r: (1, r, 0)),
            pl.BlockSpec((D, D), lambda r: (0, 0)),
            pl.BlockSpec((1, D), lambda r: (0, 0)),
        ],
        out_specs=pl.BlockSpec((rows, D), lambda r: (r, 0)),
        out_shape=jax.ShapeDtypeStruct((N, D), jnp.float32),
    )(agg, degs, W.T, b.reshape(1, D))
    return out


# speedup vs baseline: 1.0416x; 1.0018x over previous
import functools

import jax
import jax.numpy as jnp
from jax import lax
from jax.experimental import pallas as pl
from jax.experimental.pallas import tpu as pltpu
from jax.experimental.pallas import tpu_sc as plsc

N_NODES = 10000
D = 256
DH = 128
EPS = 1e-5

NC = 2
NS = 16
CHUNK = 64
NCH = 160
SEC = 32
NSEC = NCH // SEC
NSLOT = 4
EPT = CHUNK * NCH
E_PAD = EPT * NS
ACC_ROWS = 10112
ROWS_PT = ACC_ROWS // NS

ACH_CHUNK = 128
ACH = 80

_mesh = plsc.VectorSubcoreMesh(core_axis_name="c", subcore_axis_name="s",
                               num_cores=NC, num_subcores=NS)


def _degrees_body(eh_ref, zero_ref, out_ref, idx_v, vals_v, hist_sh, sem):
    c = lax.axis_index("c")
    s = lax.axis_index("s")
    ones16 = jnp.ones((16,), jnp.float32)
    for r in range(ACH_CHUNK):
        vals_v[r, :] = ones16
    pltpu.sync_copy(zero_ref.at[pl.ds(s * ROWS_PT, ROWS_PT)],
                    hist_sh.at[pl.ds(s * ROWS_PT, ROWS_PT)])
    plsc.subcore_barrier()
    pltpu.sync_copy(eh_ref.at[c, s], idx_v)

    def group(g, carry):
        descs = []
        for k in range(8):
            descs.append(
                pltpu.async_copy(vals_v, hist_sh.at[idx_v.at[g * 8 + k]],
                                 sem, add=True))
        for d in descs:
            d.wait()
        return carry

    lax.fori_loop(0, ACH // 8, group, 0)
    plsc.subcore_barrier()
    pltpu.sync_copy(hist_sh.at[pl.ds(s * ROWS_PT, ROWS_PT)],
                    out_ref.at[c, pl.ds(s * ROWS_PT, ROWS_PT)])


_degrees_kernel = pl.kernel(
    _degrees_body,
    out_type=jax.ShapeDtypeStruct((NC, ACC_ROWS, 16), jnp.float32),
    mesh=_mesh,
    scratch_types=[
        pltpu.VMEM((ACH, ACH_CHUNK), jnp.int32),
        pltpu.VMEM((ACH_CHUNK, 16), jnp.float32),
        pltpu.VMEM_SHARED((ACC_ROWS, 16), jnp.float32),
        pltpu.SemaphoreType.DMA,
    ],
    compiler_params=pltpu.CompilerParams(use_tc_tiling_on_sc=False),
)


def _ln_scale_body(h_ref, deg_ref, g_ref, b_ref, out_ref):
    h = h_ref[...]
    mean = jnp.mean(h, axis=-1, keepdims=True)
    var = jnp.mean((h - mean) ** 2, axis=-1, keepdims=True)
    hn = (h - mean) * lax.rsqrt(var + EPS) * g_ref[...] + b_ref[...]
    deg = deg_ref[0, :, 0:1]
    ns = jnp.where(deg > 0, lax.rsqrt(jnp.maximum(deg, 1.0)), 0.0)
    sc = hn * ns
    out_ref[0, :, :] = sc[:, :DH]
    out_ref[1, :, :] = sc[:, DH:]


def _edge_body(tab_ref, src_ref, dst_ref, out_ref,
               sidx_v, didx_v, rows_v, zb_v, acc_sh, gsem, ssem, zsem):
    c = lax.axis_index("c")
    s = lax.axis_index("s")

    z16 = jnp.zeros((16,), jnp.float32)

    def zrow(rr, carry):
        for l in range(DH // 16):
            zb_v[rr, pl.ds(l * 16, 16)] = z16
        return carry

    lax.fori_loop(0, 16, zrow, 0)

    zd = []
    for t in range(39):
        zd.append(pltpu.async_copy(
            zb_v, acc_sh.at[pl.ds(s * ROWS_PT + t * 16, 16)], zsem))
    zd.append(pltpu.async_copy(
        zb_v.at[pl.ds(0, 8)],
        acc_sh.at[pl.ds(s * ROWS_PT + 624, 8)], zsem))
    pltpu.sync_copy(src_ref.at[c, s], sidx_v)
    for d in zd:
        d.wait()
    plsc.subcore_barrier()

    def gather(j, slot):
        pltpu.async_copy(tab_ref.at[sidx_v.at[j]], rows_v.at[slot], gsem)

    def gwait(j, slot):
        pltpu.make_async_copy(tab_ref.at[sidx_v.at[j]],
                              rows_v.at[slot], gsem).wait()

    def swait(l, slot):
        pltpu.make_async_copy(rows_v.at[slot],
                              acc_sh.at[didx_v.at[l]], ssem).wait()

    for k in range(NSLOT):
        gather(k, k)

    def section(sec, carry):
        @pl.when(sec > 0)
        def _():
            swait(0, NSLOT - 1)
            gather(sec * SEC + NSLOT - 1, NSLOT - 1)

        pltpu.sync_copy(dst_ref.at[s, pl.ds(sec * SEC, SEC)], didx_v)

        def step(ii, carry2):
            for b in range(NSLOT):
                l = ii * NSLOT + b
                j = sec * SEC + l
                gwait(j, b)
                pltpu.async_copy(rows_v.at[b], acc_sh.at[didx_v.at[l]],
                                 ssem, add=True)
                if not (b == 0):
                    @pl.when(j + NSLOT - 1 < NCH)
                    def _():
                        swait(l - 1, b - 1)
                        gather(j + NSLOT - 1, b - 1)
                else:
                    @pl.when(jnp.logical_and(l > 0, j + NSLOT - 1 < NCH))
                    def _():
                        swait(l - 1, NSLOT - 1)
                        gather(j + NSLOT - 1, NSLOT - 1)
            return carry2

        lax.fori_loop(0, SEC // NSLOT, step, 0)
        return carry

    lax.fori_loop(0, NSEC, section, 0)
    for k in range(NSLOT):
        swait(SEC - NSLOT + k, k)
    plsc.subcore_barrier()
    pltpu.sync_copy(acc_sh.at[pl.ds(s * ROWS_PT, ROWS_PT)],
                    out_ref.at[c, pl.ds(s * ROWS_PT, ROWS_PT)])


_edge_kernel = pl.kernel(
    _edge_body,
    out_type=jax.ShapeDtypeStruct((NC, ACC_ROWS, DH), jnp.float32),
    mesh=_mesh,
    scratch_types=[
        pltpu.VMEM((NCH, CHUNK), jnp.int32),
        pltpu.VMEM((SEC, CHUNK), jnp.int32),
        pltpu.VMEM((NSLOT, CHUNK, DH), jnp.float32),
        pltpu.VMEM((16, DH), jnp.float32),
        pltpu.VMEM_SHARED((ACC_ROWS, DH), jnp.float32),
        pltpu.SemaphoreType.DMA,
        pltpu.SemaphoreType.DMA,
        pltpu.SemaphoreType.DMA,
    ],
    compiler_params=pltpu.CompilerParams(use_tc_tiling_on_sc=False),
)


def _ffn_body(agg_ref, deg_ref, wt_ref, b_ref, out_ref):
    deg = deg_ref[0, :, 0:1]
    nd = jnp.where(deg > 0, lax.rsqrt(jnp.maximum(deg, 1.0)), 0.0)
    hc = jnp.concatenate([agg_ref[0, :, :] * nd, agg_ref[1, :, :] * nd],
                         axis=-1)
    out_ref[...] = jnp.dot(hc, wt_ref[...],
                           preferred_element_type=jnp.float32) + b_ref[...]


def kernel(h, edge_index, gamma, beta, W, b):
    N = h.shape[0]
    E = edge_index.shape[1]
    pad = E_PAD - E

    src = edge_index[0]
    dst = edge_index[1]

    eh = jnp.concatenate(
        [edge_index, jnp.full((2, pad), N, jnp.int32)], axis=1
    ).reshape(NC, NS, ACH, ACH_CHUNK)
    zero16 = jnp.zeros((ACC_ROWS, 16), jnp.float32)
    degs = _degrees_kernel(eh, zero16)

    nb = 10
    rows = N // nb
    hn2 = pl.pallas_call(
        _ln_scale_body,
        grid=(nb,),
        in_specs=[
            pl.BlockSpec((rows, D), lambda r: (r, 0)),
            pl.BlockSpec((1, rows, 16), lambda r: (0, r, 0)),
            pl.BlockSpec((1, D), lambda r: (0, 0)),
            pl.BlockSpec((1, D), lambda r: (0, 0)),
        ],
        out_specs=pl.BlockSpec((NC, rows, DH), lambda r: (0, r, 0)),
        out_shape=jax.ShapeDtypeStruct((NC, N, DH), jnp.float32),
    )(h, degs, gamma.reshape(1, D), beta.reshape(1, D))

    src_p = jnp.concatenate([src, jnp.zeros((pad,), jnp.int32)])
    dst_p = jnp.concatenate([dst, jnp.full((pad,), N, jnp.int32)])
    offs = jnp.arange(NC, dtype=jnp.int32).reshape(NC, 1) * N
    srcC = (src_p[None, :] + offs).reshape(NC, NS, NCH, CHUNK)
    dstC = dst_p.reshape(NS, NCH, CHUNK)
    agg = _edge_kernel(hn2.reshape(NC * N, DH), srcC, dstC)

    out = pl.pallas_call(
        _ffn_body,
        grid=(nb,),
        in_specs=[
            pl.BlockSpec((NC, rows, DH), lambda r: (0, r, 0)),
            pl.BlockSpec((1, rows, 16), lambda r: (1, r, 0)),
            pl.BlockSpec((D, D), lambda r: (0, 0)),
            pl.BlockSpec((1, D), lambda r: (0, 0)),
        ],
        out_specs=pl.BlockSpec((rows, D), lambda r: (r, 0)),
        out_shape=jax.ShapeDtypeStruct((N, D), jnp.float32),
    )(agg, degs, W.T, b.reshape(1, D))
    return out
